# baseline (speedup 1.0000x reference)
import jax
import jax.numpy as jnp
from jax.experimental import pallas as pl


_ROWS = 256


def _mix_body(e_ref, w_ref, o_ref):
    acc = e_ref[0] * w_ref[:, 0:1]
    for k in range(1, e_ref.shape[0]):
        acc = acc + e_ref[k] * w_ref[:, k : k + 1]
    o_ref[...] = acc


def kernel(hidden_states, expert_outputs, expert_weights, expert_indices):
    K, B, S, H = expert_outputs.shape
    N = B * S
    e = expert_outputs.reshape(K, N, H)
    w = expert_weights.reshape(N, K)

    out = pl.pallas_call(
        _mix_body,
        grid=(N // _ROWS,),
        in_specs=[
            pl.BlockSpec((K, _ROWS, H), lambda i: (0, i, 0)),
            pl.BlockSpec((_ROWS, K), lambda i: (i, 0)),
        ],
        out_specs=pl.BlockSpec((_ROWS, H), lambda i: (i, 0)),
        out_shape=jax.ShapeDtypeStruct((N, H), jnp.float32),
    )(e, w)
    return out.reshape(B, S, H)


# speedup vs baseline: 1.0290x; 1.0290x over previous
import jax
import jax.numpy as jnp
from jax.experimental import pallas as pl


_ROWS = 512


def _mix_body(e_ref, w_ref, o_ref):
    acc = e_ref[0] * w_ref[:, 0:1]
    for k in range(1, e_ref.shape[0]):
        acc = acc + e_ref[k] * w_ref[:, k : k + 1]
    o_ref[...] = acc


def kernel(hidden_states, expert_outputs, expert_weights, expert_indices):
    K, B, S, H = expert_outputs.shape
    N = B * S
    e = expert_outputs.reshape(K, N, H)
    w = expert_weights.reshape(N, K)

    out = pl.pallas_call(
        _mix_body,
        grid=(N // _ROWS,),
        in_specs=[
            pl.BlockSpec((K, _ROWS, H), lambda i: (0, i, 0)),
            pl.BlockSpec((_ROWS, K), lambda i: (i, 0)),
        ],
        out_specs=pl.BlockSpec((_ROWS, H), lambda i: (i, 0)),
        out_shape=jax.ShapeDtypeStruct((N, H), jnp.float32),
    )(e, w)
    return out.reshape(B, S, H)
